# Initial kernel scaffold; baseline (speedup 1.0000x reference)
import jax
import jax.numpy as jnp
from jax.experimental import pallas as pl


def kernel(in_feat, edge_index, W1, b1, W2, b2, W3, b3, Wd, bd):
    raise NotImplementedError("write your pallas kernel here")


# speedup vs baseline: 7.0664x; 7.0664x over previous
import functools

import jax
import jax.numpy as jnp
from jax import lax
from jax.experimental import pallas as pl
from jax.experimental.pallas import tpu as pltpu
from jax.experimental.pallas import tpu_sc as plsc

N = 10000
E = 320000
D_IN = 128
H1, H2, H3 = 64, 128, 64
C = 10

CH = 128
NCHUNK = E // CH
NC, NS = 2, 16
NW = NC * NS
CH_PER_W = -(-NCHUNK // NW)
NPAD = 10240
RPT = NPAD // NS
ZR = 64

BN = 1000
GRID = N // BN

_sc_mesh = plsc.VectorSubcoreMesh(core_axis_name="c", subcore_axis_name="s")
_sc_params = pltpu.CompilerParams(use_tc_tiling_on_sc=False)


@functools.partial(
    pl.kernel,
    out_type=tuple(jax.ShapeDtypeStruct((NPAD,), jnp.float32) for _ in range(4)),
    mesh=_sc_mesh,
    scratch_types=(
        pltpu.VMEM((CH,), jnp.int32),
        pltpu.VMEM((CH,), jnp.int32),
        pltpu.VMEM((CH,), jnp.float32),
        pltpu.VMEM((RPT,), jnp.float32),
        pltpu.VMEM_SHARED((NPAD,), jnp.float32),
        pltpu.VMEM_SHARED((NPAD,), jnp.float32),
        pltpu.SemaphoreType.DMA,
    ),
    compiler_params=_sc_params,
)
def _sc_degrees(src1, dst1, do0, do1, di0, di1, src_v, dst_v, ones_v, zero_v,
                do_sp, di_sp, sem):
    c = lax.axis_index("c")
    s = lax.axis_index("s")
    w = s * NC + c
    for k in range(CH // 16):
        ones_v[pl.ds(k * 16, 16)] = jnp.ones((16,), jnp.float32)
    for k in range(RPT // 16):
        zero_v[pl.ds(k * 16, 16)] = jnp.zeros((16,), jnp.float32)
    pltpu.sync_copy(zero_v, do_sp.at[pl.ds(s * RPT, RPT)])
    pltpu.sync_copy(zero_v, di_sp.at[pl.ds(s * RPT, RPT)])
    plsc.subcore_barrier()

    def body(j, carry):
        r = j * NW + w

        @pl.when(r < NCHUNK)
        def _():
            pltpu.sync_copy(src1.at[pl.ds(r * CH, CH)], src_v)
            pltpu.sync_copy(dst1.at[pl.ds(r * CH, CH)], dst_v)
            pltpu.sync_copy(ones_v, do_sp.at[src_v], add=True)
            pltpu.sync_copy(ones_v, di_sp.at[dst_v], add=True)
        return carry

    lax.fori_loop(0, CH_PER_W, body, 0)
    plsc.subcore_barrier()

    @pl.when(c == 0)
    def _():
        pltpu.sync_copy(do_sp.at[pl.ds(s * RPT, RPT)], do0.at[pl.ds(s * RPT, RPT)])
        pltpu.sync_copy(di_sp.at[pl.ds(s * RPT, RPT)], di0.at[pl.ds(s * RPT, RPT)])

    @pl.when(c == 1)
    def _():
        pltpu.sync_copy(do_sp.at[pl.ds(s * RPT, RPT)], do1.at[pl.ds(s * RPT, RPT)])
        pltpu.sync_copy(di_sp.at[pl.ds(s * RPT, RPT)], di1.at[pl.ds(s * RPT, RPT)])


@functools.partial(
    pl.kernel,
    out_type=tuple(jax.ShapeDtypeStruct((NPAD, H1), jnp.float32) for _ in range(2)),
    mesh=_sc_mesh,
    scratch_types=(
        pltpu.VMEM((CH,), jnp.int32),
        pltpu.VMEM((CH,), jnp.int32),
        pltpu.VMEM((CH, H1), jnp.float32),
        pltpu.VMEM((ZR, H1), jnp.float32),
        pltpu.VMEM_SHARED((NPAD, H1), jnp.float32),
        pltpu.SemaphoreType.DMA,
    ),
    compiler_params=_sc_params,
)
def _sc_aggregate(y, src1, dst1, z0, z1, src_v, dst_v, rows_v, zero_v, z_sp, sem):
    c = lax.axis_index("c")
    s = lax.axis_index("s")
    w = s * NC + c
    for a in range(ZR):
        for b in range(H1 // 16):
            zero_v[a, pl.ds(b * 16, 16)] = jnp.zeros((16,), jnp.float32)
    for k in range(RPT // ZR):
        pltpu.sync_copy(zero_v, z_sp.at[pl.ds(s * RPT + k * ZR, ZR)])
    plsc.subcore_barrier()

    def body(j, carry):
        r = j * NW + w

        @pl.when(r < NCHUNK)
        def _():
            pltpu.sync_copy(src1.at[pl.ds(r * CH, CH)], src_v)
            pltpu.sync_copy(dst1.at[pl.ds(r * CH, CH)], dst_v)
            pltpu.async_copy(y.at[src_v], rows_v, sem).wait()
            pltpu.sync_copy(rows_v, z_sp.at[dst_v], add=True)
        return carry

    lax.fori_loop(0, CH_PER_W, body, 0)
    plsc.subcore_barrier()

    @pl.when(c == 0)
    def _():
        pltpu.sync_copy(z_sp.at[pl.ds(s * RPT, RPT)], z0.at[pl.ds(s * RPT, RPT)])

    @pl.when(c == 1)
    def _():
        pltpu.sync_copy(z_sp.at[pl.ds(s * RPT, RPT)], z1.at[pl.ds(s * RPT, RPT)])


def _tc_l1_body(x_ref, w1_ref, deg_ref, y_ref, no_ref, ni_ref):
    deg_o = deg_ref[:, 0:1] + deg_ref[:, 1:2]
    deg_i = deg_ref[:, 2:3] + deg_ref[:, 3:4]
    no = lax.rsqrt(jnp.maximum(deg_o, 1.0))
    ni = lax.rsqrt(jnp.maximum(deg_i, 1.0))
    y_ref[...] = jnp.dot(x_ref[...], w1_ref[...],
                         preferred_element_type=jnp.float32) * no
    no_ref[...] = no
    ni_ref[...] = ni


_tc_l1 = pl.pallas_call(
    _tc_l1_body,
    grid=(GRID,),
    in_specs=[
        pl.BlockSpec((BN, D_IN), lambda i: (i, 0)),
        pl.BlockSpec((D_IN, H1), lambda i: (0, 0)),
        pl.BlockSpec((BN, 4), lambda i: (i, 0)),
    ],
    out_specs=[
        pl.BlockSpec((BN, H1), lambda i: (i, 0)),
        pl.BlockSpec((BN, 1), lambda i: (i, 0)),
        pl.BlockSpec((BN, 1), lambda i: (i, 0)),
    ],
    out_shape=[
        jax.ShapeDtypeStruct((N, H1), jnp.float32),
        jax.ShapeDtypeStruct((N, 1), jnp.float32),
        jax.ShapeDtypeStruct((N, 1), jnp.float32),
    ],
)


def _tc_mid1_body(z0_ref, z1_ref, ni_ref, no_ref, b1_ref, out_ref):
    z = z0_ref[...] + z1_ref[...]
    out_ref[...] = jnp.maximum(z * ni_ref[...] + b1_ref[...], 0.0) * no_ref[...]


_tc_mid1 = pl.pallas_call(
    _tc_mid1_body,
    grid=(GRID,),
    in_specs=[
        pl.BlockSpec((BN, H1), lambda i: (i, 0)),
        pl.BlockSpec((BN, H1), lambda i: (i, 0)),
        pl.BlockSpec((BN, 1), lambda i: (i, 0)),
        pl.BlockSpec((BN, 1), lambda i: (i, 0)),
        pl.BlockSpec((1, H1), lambda i: (0, 0)),
    ],
    out_specs=pl.BlockSpec((BN, H1), lambda i: (i, 0)),
    out_shape=jax.ShapeDtypeStruct((N, H1), jnp.float32),
)


def _tc_mid2_body(z0_ref, z1_ref, ni_ref, no_ref, w2_ref, b2_ref, w3_ref, out_ref):
    z = (z0_ref[...] + z1_ref[...]) * ni_ref[...]
    t = jnp.maximum(jnp.dot(z, w2_ref[...],
                            preferred_element_type=jnp.float32) + b2_ref[...], 0.0)
    out_ref[...] = jnp.dot(t, w3_ref[...],
                           preferred_element_type=jnp.float32) * no_ref[...]


_tc_mid2 = pl.pallas_call(
    _tc_mid2_body,
    grid=(GRID,),
    in_specs=[
        pl.BlockSpec((BN, H1), lambda i: (i, 0)),
        pl.BlockSpec((BN, H1), lambda i: (i, 0)),
        pl.BlockSpec((BN, 1), lambda i: (i, 0)),
        pl.BlockSpec((BN, 1), lambda i: (i, 0)),
        pl.BlockSpec((H1, H2), lambda i: (0, 0)),
        pl.BlockSpec((1, H2), lambda i: (0, 0)),
        pl.BlockSpec((H2, H3), lambda i: (0, 0)),
    ],
    out_specs=pl.BlockSpec((BN, H3), lambda i: (i, 0)),
    out_shape=jax.ShapeDtypeStruct((N, H3), jnp.float32),
)


def _tc_final_body(z0_ref, z1_ref, ni_ref, b3_ref, wd_ref, bd_ref, out_ref, acc_ref):
    i = pl.program_id(0)

    @pl.when(i == 0)
    def _():
        acc_ref[...] = jnp.zeros_like(acc_ref)

    h3 = jnp.maximum((z0_ref[...] + z1_ref[...]) * ni_ref[...] + b3_ref[...], 0.0)
    acc_ref[...] += jnp.sum(h3, axis=0, keepdims=True)

    @pl.when(i == pl.num_programs(0) - 1)
    def _():
        out_ref[...] = jnp.dot(jnp.tanh(acc_ref[...]), wd_ref[...],
                               preferred_element_type=jnp.float32) + bd_ref[...]


_tc_final = pl.pallas_call(
    _tc_final_body,
    grid=(GRID,),
    in_specs=[
        pl.BlockSpec((BN, H3), lambda i: (i, 0)),
        pl.BlockSpec((BN, H3), lambda i: (i, 0)),
        pl.BlockSpec((BN, 1), lambda i: (i, 0)),
        pl.BlockSpec((1, H3), lambda i: (0, 0)),
        pl.BlockSpec((H3, C), lambda i: (0, 0)),
        pl.BlockSpec((1, C), lambda i: (0, 0)),
    ],
    out_specs=pl.BlockSpec((1, C), lambda i: (0, 0)),
    out_shape=jax.ShapeDtypeStruct((1, C), jnp.float32),
    scratch_shapes=[pltpu.VMEM((1, H3), jnp.float32)],
)


def kernel(in_feat, edge_index, W1, b1, W2, b2, W3, b3, Wd, bd):
    src1 = edge_index[0]
    dst1 = edge_index[1]

    do0, do1, di0, di1 = _sc_degrees(src1, dst1)
    deg4 = jnp.stack([do0, do1, di0, di1], axis=1)
    y1, no, ni = _tc_l1(in_feat, W1, deg4)
    z0, z1 = _sc_aggregate(y1, src1, dst1)
    h1s = _tc_mid1(z0, z1, ni, no, b1.reshape(1, H1))
    z0, z1 = _sc_aggregate(h1s, src1, dst1)
    y3 = _tc_mid2(z0, z1, ni, no, W2, b2.reshape(1, H2), W3)
    z0, z1 = _sc_aggregate(y3, src1, dst1)
    return _tc_final(z0, z1, ni, b3.reshape(1, H3), Wd, bd.reshape(1, C))
